# Initial kernel scaffold; baseline (speedup 1.0000x reference)
import jax
import jax.numpy as jnp
from jax.experimental import pallas as pl


def kernel(hidden_states, gate_w, w_gate, w_up, w_down):
    raise NotImplementedError("write your pallas kernel here")


# speedup vs baseline: 1.7054x; 1.7054x over previous
import functools

import jax
import jax.numpy as jnp
from jax.experimental import pallas as pl
from jax.experimental.pallas import tpu as pltpu

E = 16
K = 2
D = 1024
F = 1024
T = 2048

B = 256
NB = (T * K) // B
NPAIR = NB + E - 1


def _moe_ffn_kernel(
    blk_expert_ref, blk_row_ref, blk_first_ref,
    x_ref, e_ref, w_ref, wg_ref, wu_ref, wd_ref,
    out_ref,
):
    i = pl.program_id(0)
    be = blk_expert_ref[i]

    x = x_ref[...]
    g = jnp.dot(x, wg_ref[0], preferred_element_type=jnp.float32)
    u = jnp.dot(x, wu_ref[0], preferred_element_type=jnp.float32)
    h = (jax.nn.silu(g) * u).astype(jnp.bfloat16)
    y = jnp.dot(h, wd_ref[0], preferred_element_type=jnp.float32)

    coef = jnp.where(e_ref[0, 0, :] == be, w_ref[0, 0, :], 0.0)
    y = y * coef[:, None]

    @pl.when(blk_first_ref[i] == 1)
    def _():
        out_ref[...] = y

    @pl.when(blk_first_ref[i] == 0)
    def _():
        out_ref[...] += y


def _grouped_ffn(x_sorted, e_sorted, w_sorted, wg, wu, wd,
                 blk_expert, blk_row, blk_first):
    grid_spec = pltpu.PrefetchScalarGridSpec(
        num_scalar_prefetch=3,
        grid=(NPAIR,),
        in_specs=[
            pl.BlockSpec((B, D), lambda i, be, br, bf: (br[i], 0)),
            pl.BlockSpec((1, 1, B), lambda i, be, br, bf: (br[i], 0, 0)),
            pl.BlockSpec((1, 1, B), lambda i, be, br, bf: (br[i], 0, 0)),
            pl.BlockSpec((1, D, F),
                         lambda i, be, br, bf: (jnp.maximum(be[i], 0), 0, 0)),
            pl.BlockSpec((1, D, F),
                         lambda i, be, br, bf: (jnp.maximum(be[i], 0), 0, 0)),
            pl.BlockSpec((1, F, D),
                         lambda i, be, br, bf: (jnp.maximum(be[i], 0), 0, 0)),
        ],
        out_specs=pl.BlockSpec((B, D), lambda i, be, br, bf: (br[i], 0)),
    )
    return pl.pallas_call(
        _moe_ffn_kernel,
        grid_spec=grid_spec,
        out_shape=jax.ShapeDtypeStruct((T * K, D), jnp.float32),
        compiler_params=pltpu.CompilerParams(
            dimension_semantics=("arbitrary",),
        ),
    )(
        blk_expert, blk_row, blk_first,
        x_sorted,
        e_sorted.reshape(NB, 1, B),
        w_sorted.reshape(NB, 1, B),
        wg, wu, wd,
    )


def kernel(hidden_states, gate_w, w_gate, w_up, w_down):
    logits = hidden_states @ gate_w
    probs = jax.nn.softmax(logits, axis=-1)
    topk_w, topk_idx = jax.lax.top_k(probs, K)
    topk_w = topk_w / jnp.sum(topk_w, axis=-1, keepdims=True)

    e_flat = topk_idx.reshape(-1).astype(jnp.int32)
    w_flat = topk_w.reshape(-1)
    sort_idx = jnp.argsort(e_flat, stable=True)
    e_sorted = e_flat[sort_idx]
    w_sorted = w_flat[sort_idx]
    tok_sorted = (sort_idx // K).astype(jnp.int32)

    first = e_sorted[0::B]
    last = e_sorted[B - 1::B]
    span = last - first + 1
    pair_start = jnp.concatenate(
        [jnp.zeros((1,), jnp.int32), jnp.cumsum(span)[:-1].astype(jnp.int32)])
    total = pair_start[-1] + span[-1]
    j = jnp.arange(NPAIR, dtype=jnp.int32)
    b_of = (jnp.searchsorted(pair_start, j, side="right") - 1).astype(jnp.int32)
    be = first[b_of] + (j - pair_start[b_of])
    valid = j < total
    blk_expert = jnp.where(valid, be, -1).astype(jnp.int32)
    blk_row = b_of
    blk_first = (valid & (j == pair_start[b_of])).astype(jnp.int32)

    x_sorted = hidden_states[tok_sorted].astype(jnp.bfloat16)
    wg = w_gate.astype(jnp.bfloat16)
    wu = w_up.astype(jnp.bfloat16)
    wd = w_down.astype(jnp.bfloat16)

    y_sorted = _grouped_ffn(x_sorted, e_sorted, w_sorted, wg, wu, wd,
                            blk_expert, blk_row, blk_first)

    inv = jnp.zeros((T * K,), jnp.int32).at[sort_idx].set(
        jnp.arange(T * K, dtype=jnp.int32))
    out = y_sorted[inv].reshape(T, K, D).sum(axis=1)
    return out
